# baseline (speedup 1.0000x reference)
import functools

import jax
import jax.numpy as jnp
from jax import lax
from jax.experimental import pallas as pl
from jax.experimental.pallas import tpu as pltpu
from jax.experimental.pallas import tpu_sc as plsc

_LANES = 16
_NUM_CORES = 2
_NUM_SUBCORES = 16
_NUM_WORKERS = _NUM_CORES * _NUM_SUBCORES


def _mf_body(uid_hbm, iid_hbm, uemb_hbm, iemb_hbm, ubias_hbm, ibias_hbm,
             gbias_hbm, out_hbm,
             uid_v, iid_v, urows_v, irows_v, ubias_v, ibias_v, gb_v, out_v,
             sem):
    d_dim = uemb_hbm.shape[1]
    b_per_w = uid_v.shape[0]
    wid = lax.axis_index("s") * _NUM_CORES + lax.axis_index("c")
    base = wid * b_per_w

    pltpu.sync_copy(uid_hbm.at[pl.ds(base, b_per_w)], uid_v)
    pltpu.sync_copy(iid_hbm.at[pl.ds(base, b_per_w)], iid_v)
    cp_u = pltpu.async_copy(uemb_hbm.at[uid_v], urows_v, sem)
    cp_i = pltpu.async_copy(iemb_hbm.at[iid_v], irows_v, sem)
    cp_ub = pltpu.async_copy(ubias_hbm.at[uid_v], ubias_v, sem)
    cp_ib = pltpu.async_copy(ibias_hbm.at[iid_v], ibias_v, sem)
    pltpu.sync_copy(gbias_hbm, gb_v)
    cp_u.wait()
    cp_i.wait()
    cp_ub.wait()
    cp_ib.wait()

    lane = lax.iota(jnp.int32, _LANES)
    gb = gb_v[...]

    def group_body(g, carry):
        acc0 = (ubias_v[pl.ds(g * _LANES, _LANES)]
                + ibias_v[pl.ds(g * _LANES, _LANES)]
                + gb)
        row = g * _LANES + lane

        def d_body(d, acc):
            col = jnp.full((_LANES,), d, jnp.int32)
            gu = plsc.load_gather(urows_v, [row, col])
            gi = plsc.load_gather(irows_v, [row, col])
            return acc + gu * gi

        acc = lax.fori_loop(0, d_dim, d_body, acc0)
        out_v[pl.ds(g * _LANES, _LANES)] = acc
        return carry

    lax.fori_loop(0, b_per_w // _LANES, group_body, 0)
    pltpu.sync_copy(out_v, out_hbm.at[pl.ds(base, b_per_w)])


def kernel(user_ids, item_ids, user_emb_w, item_emb_w, user_bias_w,
           item_bias_w, global_bias):
    batch = user_ids.shape[0]
    d_dim = user_emb_w.shape[1]
    b_per_w = batch // _NUM_WORKERS

    mesh = plsc.VectorSubcoreMesh(core_axis_name="c", subcore_axis_name="s",
                                  num_cores=_NUM_CORES,
                                  num_subcores=_NUM_SUBCORES)
    k = pl.kernel(
        _mf_body,
        out_type=jax.ShapeDtypeStruct((batch,), jnp.float32),
        mesh=mesh,
        compiler_params=pltpu.CompilerParams(needs_layout_passes=False,
                                             use_tc_tiling_on_sc=False),
        scratch_types=[
            pltpu.VMEM((b_per_w,), jnp.int32),
            pltpu.VMEM((b_per_w,), jnp.int32),
            pltpu.VMEM((b_per_w, d_dim), jnp.float32),
            pltpu.VMEM((b_per_w, d_dim), jnp.float32),
            pltpu.VMEM((b_per_w,), jnp.float32),
            pltpu.VMEM((b_per_w,), jnp.float32),
            pltpu.VMEM((_LANES,), jnp.float32),
            pltpu.VMEM((b_per_w,), jnp.float32),
            pltpu.SemaphoreType.DMA,
        ],
    )
    gbias_lanes = jnp.broadcast_to(global_bias, (_LANES,))
    return k(user_ids.astype(jnp.int32), item_ids.astype(jnp.int32),
             user_emb_w, item_emb_w,
             user_bias_w.reshape(-1), item_bias_w.reshape(-1), gbias_lanes)
